# baseline (speedup 1.0000x reference)
import functools

import jax
import jax.numpy as jnp
from jax import lax
from jax.experimental import pallas as pl
from jax.experimental.pallas import tpu as pltpu
from jax.experimental.pallas import tpu_sc as plsc

N = 10000
D = 128
G = 64

CH = 128
PH = 32
PHASES0 = 4
PHASES1 = 1
TOTCH = 16 * PH * (PHASES0 + PHASES1)
EP = TOTCH * CH
NACC = 10240
ZROWS = 640
NBUF = 2


@functools.cache
def _get_sc_aggregate():
    mesh = plsc.VectorSubcoreMesh(core_axis_name="c", subcore_axis_name="s")

    @functools.partial(
        pl.kernel,
        out_type=jax.ShapeDtypeStruct((2, NACC, D), jnp.float32),
        scratch_types=[
            pltpu.VMEM((PH, CH), jnp.int32),
            pltpu.VMEM((PH, CH), jnp.int32),
            pltpu.VMEM((NBUF, CH, D), jnp.float32),
            pltpu.VMEM_SHARED((NACC, D), jnp.float32),
        ] + [pltpu.SemaphoreType.DMA] * NBUF,
        mesh=mesh,
    )
    def _sc_aggregate(x_hbm, srcp_hbm, dstp_hbm, zeros_hbm, out_hbm,
                      src_v, dst_v, rows_v, acc, *sems):
        c = lax.axis_index("c")
        s = lax.axis_index("s")

        pltpu.sync_copy(zeros_hbm, acc.at[pl.ds(s * ZROWS, ZROWS)])
        plsc.subcore_barrier()

        def run_phase(base):
            pltpu.sync_copy(srcp_hbm.at[pl.ds(base, PH)], src_v)
            pltpu.sync_copy(dstp_hbm.at[pl.ds(base, PH)], dst_v)

            for b in range(NBUF):
                pltpu.async_copy(x_hbm.at[src_v.at[b]], rows_v.at[b], sems[b])

            def ring_body(t, carry):
                for b in range(NBUF):
                    ci = t * NBUF + b
                    pltpu.make_async_copy(x_hbm.at[src_v.at[ci]],
                                          rows_v.at[b], sems[b]).wait()
                    pltpu.sync_copy(rows_v.at[b], acc.at[dst_v.at[ci]],
                                    add=True)
                    pltpu.async_copy(x_hbm.at[src_v.at[ci + NBUF]],
                                     rows_v.at[b], sems[b])
                return carry

            lax.fori_loop(0, PH // NBUF - 1, ring_body, 0)
            for b in range(NBUF):
                ci = PH - NBUF + b
                pltpu.make_async_copy(x_hbm.at[src_v.at[ci]], rows_v.at[b],
                                      sems[b]).wait()
                pltpu.sync_copy(rows_v.at[b], acc.at[dst_v.at[ci]], add=True)

        @pl.when(c == 0)
        def _():
            for p in range(PHASES0):
                run_phase(16 * PH * PHASES1 + s * PH * PHASES0 + p * PH)

        @pl.when(c == 1)
        def _():
            for p in range(PHASES1):
                run_phase(s * PH * PHASES1 + p * PH)

        plsc.subcore_barrier()

        pltpu.sync_copy(acc.at[pl.ds(s * ZROWS, ZROWS)],
                        out_hbm.at[c, pl.ds(s * ZROWS, ZROWS)])

    return _sc_aggregate


ROWS1 = 1000


def _mlp1_body(eps_ref, x_ref, p0_ref, p1_ref, w1_ref, b1_ref, h_ref, st_ref):
    i = pl.program_id(0)
    a = x_ref[...] * (1.0 + eps_ref[0, 0]) + p0_ref[...] + p1_ref[...]
    h = jnp.dot(a, w1_ref[...], preferred_element_type=jnp.float32) + b1_ref[...]
    h_ref[...] = h
    s = jnp.sum(h, axis=0, keepdims=True)
    s2 = jnp.sum(h * h, axis=0, keepdims=True)

    @pl.when(i == 0)
    def _():
        st_ref[0:1, :] = s
        st_ref[1:2, :] = s2

    @pl.when(i > 0)
    def _():
        st_ref[0:1, :] += s
        st_ref[1:2, :] += s2


_mlp1 = pl.pallas_call(
    _mlp1_body,
    grid=(N // ROWS1,),
    in_specs=[
        pl.BlockSpec(memory_space=pltpu.SMEM),
        pl.BlockSpec((ROWS1, D), lambda i: (i, 0)),
        pl.BlockSpec((ROWS1, D), lambda i: (i, 0)),
        pl.BlockSpec((ROWS1, D), lambda i: (i, 0)),
        pl.BlockSpec((D, D), lambda i: (0, 0)),
        pl.BlockSpec((1, D), lambda i: (0, 0)),
    ],
    out_specs=[
        pl.BlockSpec((ROWS1, D), lambda i: (i, 0)),
        pl.BlockSpec((2, D), lambda i: (0, 0)),
    ],
    out_shape=[
        jax.ShapeDtypeStruct((N, D), jnp.float32),
        jax.ShapeDtypeStruct((2, D), jnp.float32),
    ],
    compiler_params=pltpu.CompilerParams(dimension_semantics=("arbitrary",)),
)


def _mlp2_body(h_ref, st_ref, bng_ref, bnb_ref, w2_ref, b2_ref, lng_ref,
               lnb_ref, out_ref):
    m = st_ref[0:1, :] / N
    v = st_ref[1:2, :] / N - m * m
    h = (h_ref[...] - m) * lax.rsqrt(v + 1e-5) * bng_ref[...] + bnb_ref[...]
    h = jnp.maximum(h, 0.0)
    h = jnp.dot(h, w2_ref[...], preferred_element_type=jnp.float32) + b2_ref[...]
    mu = jnp.mean(h, axis=1, keepdims=True)
    va = jnp.mean(h * h, axis=1, keepdims=True) - mu * mu
    h = (h - mu) * lax.rsqrt(va + 1e-5) * lng_ref[...] + lnb_ref[...]
    out_ref[...] = jnp.where(h > 0, h, 0.1 * h)


_mlp2 = pl.pallas_call(
    _mlp2_body,
    grid=(N // ROWS1,),
    in_specs=[
        pl.BlockSpec((ROWS1, D), lambda i: (i, 0)),
        pl.BlockSpec((2, D), lambda i: (0, 0)),
        pl.BlockSpec((1, D), lambda i: (0, 0)),
        pl.BlockSpec((1, D), lambda i: (0, 0)),
        pl.BlockSpec((D, D), lambda i: (0, 0)),
        pl.BlockSpec((1, D), lambda i: (0, 0)),
        pl.BlockSpec((1, D), lambda i: (0, 0)),
        pl.BlockSpec((1, D), lambda i: (0, 0)),
    ],
    out_specs=pl.BlockSpec((ROWS1, D), lambda i: (i, 0)),
    out_shape=jax.ShapeDtypeStruct((N, D), jnp.float32),
    compiler_params=pltpu.CompilerParams(dimension_semantics=("arbitrary",)),
)


ROWSP = 200


def _pool_body(x_ref, b_ref, wf_ref, bf_ref, out_ref, mx, sm, cnt):
    i = pl.program_id(0)

    @pl.when(i == 0)
    def _():
        mx[...] = jnp.full((G, D), -jnp.inf, jnp.float32)
        sm[...] = jnp.zeros((G, D), jnp.float32)
        cnt[...] = jnp.zeros((G, 1), jnp.float32)

    x = x_ref[...]
    gids = lax.broadcasted_iota(jnp.int32, (1, G), 1)
    oh = (b_ref[...] == gids).astype(jnp.float32)
    dn = (((0,), (0,)), ((), ()))
    sm[...] += lax.dot_general(oh, x, dn, preferred_element_type=jnp.float32)
    cnt[...] += lax.dot_general(oh, jnp.ones((ROWSP, 1), jnp.float32), dn,
                                preferred_element_type=jnp.float32)
    b3 = lax.broadcast_in_dim(b_ref[...], (ROWSP, G, D), (0, 1))
    g3 = lax.broadcasted_iota(jnp.int32, (ROWSP, G, D), 1)
    x3 = lax.broadcast_in_dim(x, (ROWSP, G, D), (0, 2))
    big = jnp.where(b3 == g3, x3, -jnp.inf)
    mx[...] = jnp.maximum(mx[...], jnp.max(big, axis=0))

    @pl.when(i == pl.num_programs(0) - 1)
    def _():
        mean = sm[...] / jnp.maximum(cnt[...], 1.0)
        res = jnp.dot(mx[...], wf_ref[0:D, :], preferred_element_type=jnp.float32)
        res += jnp.dot(mean, wf_ref[D:2 * D, :], preferred_element_type=jnp.float32)
        out_ref[...] = res + bf_ref[...]


_pool = pl.pallas_call(
    _pool_body,
    grid=(N // ROWSP,),
    in_specs=[
        pl.BlockSpec((ROWSP, D), lambda i: (i, 0)),
        pl.BlockSpec((ROWSP, 1), lambda i: (i, 0)),
        pl.BlockSpec((2 * D, D), lambda i: (0, 0)),
        pl.BlockSpec((1, D), lambda i: (0, 0)),
    ],
    out_specs=pl.BlockSpec((G, D), lambda i: (0, 0)),
    out_shape=jax.ShapeDtypeStruct((G, D), jnp.float32),
    scratch_shapes=[
        pltpu.VMEM((G, D), jnp.float32),
        pltpu.VMEM((G, D), jnp.float32),
        pltpu.VMEM((G, 1), jnp.float32),
    ],
    compiler_params=pltpu.CompilerParams(dimension_semantics=("arbitrary",)),
)


def kernel(x, edge_index, batch, W1, b1, bn_g, bn_b, W2, b2, eps, ln_g, ln_b,
           Wf, bf):
    E = edge_index.shape[1]
    pad = EP - E
    src = jnp.concatenate([edge_index[0], jnp.zeros((pad,), jnp.int32)])
    dst = jnp.concatenate([edge_index[1], jnp.full((pad,), N, jnp.int32)])
    srcp = src.reshape(EP // CH, CH)
    dstp = dst.reshape(EP // CH, CH)
    zeros_blk = jnp.zeros((ZROWS, D), jnp.float32)
    batch2 = batch.reshape(N, 1)

    out = x
    for l in range(W1.shape[0]):
        partials = _get_sc_aggregate()(out, srcp, dstp, zeros_blk)
        h1, st = _mlp1(eps[l].reshape(1, 1), out, partials[0], partials[1],
                       W1[l], b1[l].reshape(1, D))
        out = _mlp2(h1, st, bn_g[l].reshape(1, D), bn_b[l].reshape(1, D),
                    W2[l], b2[l].reshape(1, D), ln_g[l].reshape(1, D),
                    ln_b[l].reshape(1, D))
    return _pool(out, batch2, Wf, bf.reshape(1, D))


# speedup vs baseline: 1.0022x; 1.0022x over previous
import functools

import jax
import jax.numpy as jnp
from jax import lax
from jax.experimental import pallas as pl
from jax.experimental.pallas import tpu as pltpu
from jax.experimental.pallas import tpu_sc as plsc

N = 10000
D = 128
G = 64

CH = 128
PH = 40
NPHASE = 2
CHUNKS = PH * NPHASE
EP = 32 * CHUNKS * CH
NACC = 10240
ZROWS = 640
NBUF = 2


@functools.cache
def _get_sc_aggregate():
    mesh = plsc.VectorSubcoreMesh(core_axis_name="c", subcore_axis_name="s")

    @functools.partial(
        pl.kernel,
        out_type=jax.ShapeDtypeStruct((2, NACC, D), jnp.float32),
        scratch_types=[
            pltpu.VMEM((PH, CH), jnp.int32),
            pltpu.VMEM((PH, CH), jnp.int32),
            pltpu.VMEM((NBUF, CH, D), jnp.float32),
            pltpu.VMEM_SHARED((NACC, D), jnp.float32),
        ] + [pltpu.SemaphoreType.DMA] * NBUF,
        mesh=mesh,
    )
    def _sc_aggregate(x_hbm, srcp_hbm, dstp_hbm, zeros_hbm, out_hbm,
                      src_v, dst_v, rows_v, acc, *sems):
        c = lax.axis_index("c")
        s = lax.axis_index("s")

        pltpu.sync_copy(zeros_hbm, acc.at[pl.ds(s * ZROWS, ZROWS)])
        plsc.subcore_barrier()

        def run_phase(base):
            pltpu.sync_copy(srcp_hbm.at[pl.ds(base, PH)], src_v)
            pltpu.sync_copy(dstp_hbm.at[pl.ds(base, PH)], dst_v)

            for b in range(NBUF):
                pltpu.async_copy(x_hbm.at[src_v.at[b]], rows_v.at[b], sems[b])

            def ring_body(t, carry):
                for b in range(NBUF):
                    ci = t * NBUF + b
                    pltpu.make_async_copy(x_hbm.at[src_v.at[ci]],
                                          rows_v.at[b], sems[b]).wait()
                    pltpu.sync_copy(rows_v.at[b], acc.at[dst_v.at[ci]],
                                    add=True)
                    pltpu.async_copy(x_hbm.at[src_v.at[ci + NBUF]],
                                     rows_v.at[b], sems[b])
                return carry

            lax.fori_loop(0, PH // NBUF - 1, ring_body, 0)
            for b in range(NBUF):
                ci = PH - NBUF + b
                pltpu.make_async_copy(x_hbm.at[src_v.at[ci]], rows_v.at[b],
                                      sems[b]).wait()
                pltpu.sync_copy(rows_v.at[b], acc.at[dst_v.at[ci]], add=True)

        tid = c * 16 + s
        for p in range(NPHASE):
            run_phase(tid * CHUNKS + p * PH)

        plsc.subcore_barrier()

        pltpu.sync_copy(acc.at[pl.ds(s * ZROWS, ZROWS)],
                        out_hbm.at[c, pl.ds(s * ZROWS, ZROWS)])

    return _sc_aggregate


ROWS1 = 1000


def _mlp1_body(eps_ref, x_ref, p0_ref, p1_ref, w1_ref, b1_ref, h_ref, st_ref):
    i = pl.program_id(0)
    a = x_ref[...] * (1.0 + eps_ref[0, 0]) + p0_ref[...] + p1_ref[...]
    h = jnp.dot(a, w1_ref[...], preferred_element_type=jnp.float32) + b1_ref[...]
    h_ref[...] = h
    s = jnp.sum(h, axis=0, keepdims=True)
    s2 = jnp.sum(h * h, axis=0, keepdims=True)

    @pl.when(i == 0)
    def _():
        st_ref[0:1, :] = s
        st_ref[1:2, :] = s2

    @pl.when(i > 0)
    def _():
        st_ref[0:1, :] += s
        st_ref[1:2, :] += s2


_mlp1 = pl.pallas_call(
    _mlp1_body,
    grid=(N // ROWS1,),
    in_specs=[
        pl.BlockSpec(memory_space=pltpu.SMEM),
        pl.BlockSpec((ROWS1, D), lambda i: (i, 0)),
        pl.BlockSpec((ROWS1, D), lambda i: (i, 0)),
        pl.BlockSpec((ROWS1, D), lambda i: (i, 0)),
        pl.BlockSpec((D, D), lambda i: (0, 0)),
        pl.BlockSpec((1, D), lambda i: (0, 0)),
    ],
    out_specs=[
        pl.BlockSpec((ROWS1, D), lambda i: (i, 0)),
        pl.BlockSpec((2, D), lambda i: (0, 0)),
    ],
    out_shape=[
        jax.ShapeDtypeStruct((N, D), jnp.float32),
        jax.ShapeDtypeStruct((2, D), jnp.float32),
    ],
    compiler_params=pltpu.CompilerParams(dimension_semantics=("arbitrary",)),
)


def _mlp2_body(h_ref, st_ref, bng_ref, bnb_ref, w2_ref, b2_ref, lng_ref,
               lnb_ref, out_ref):
    m = st_ref[0:1, :] / N
    v = st_ref[1:2, :] / N - m * m
    h = (h_ref[...] - m) * lax.rsqrt(v + 1e-5) * bng_ref[...] + bnb_ref[...]
    h = jnp.maximum(h, 0.0)
    h = jnp.dot(h, w2_ref[...], preferred_element_type=jnp.float32) + b2_ref[...]
    mu = jnp.mean(h, axis=1, keepdims=True)
    va = jnp.mean(h * h, axis=1, keepdims=True) - mu * mu
    h = (h - mu) * lax.rsqrt(va + 1e-5) * lng_ref[...] + lnb_ref[...]
    out_ref[...] = jnp.where(h > 0, h, 0.1 * h)


_mlp2 = pl.pallas_call(
    _mlp2_body,
    grid=(N // ROWS1,),
    in_specs=[
        pl.BlockSpec((ROWS1, D), lambda i: (i, 0)),
        pl.BlockSpec((2, D), lambda i: (0, 0)),
        pl.BlockSpec((1, D), lambda i: (0, 0)),
        pl.BlockSpec((1, D), lambda i: (0, 0)),
        pl.BlockSpec((D, D), lambda i: (0, 0)),
        pl.BlockSpec((1, D), lambda i: (0, 0)),
        pl.BlockSpec((1, D), lambda i: (0, 0)),
        pl.BlockSpec((1, D), lambda i: (0, 0)),
    ],
    out_specs=pl.BlockSpec((ROWS1, D), lambda i: (i, 0)),
    out_shape=jax.ShapeDtypeStruct((N, D), jnp.float32),
    compiler_params=pltpu.CompilerParams(dimension_semantics=("arbitrary",)),
)


ROWSP = 200


def _pool_body(x_ref, b_ref, wf_ref, bf_ref, out_ref, mx, sm, cnt):
    i = pl.program_id(0)

    @pl.when(i == 0)
    def _():
        mx[...] = jnp.full((G, D), -jnp.inf, jnp.float32)
        sm[...] = jnp.zeros((G, D), jnp.float32)
        cnt[...] = jnp.zeros((G, 1), jnp.float32)

    x = x_ref[...]
    gids = lax.broadcasted_iota(jnp.int32, (1, G), 1)
    oh = (b_ref[...] == gids).astype(jnp.float32)
    dn = (((0,), (0,)), ((), ()))
    sm[...] += lax.dot_general(oh, x, dn, preferred_element_type=jnp.float32)
    cnt[...] += lax.dot_general(oh, jnp.ones((ROWSP, 1), jnp.float32), dn,
                                preferred_element_type=jnp.float32)
    b3 = lax.broadcast_in_dim(b_ref[...], (ROWSP, G, D), (0, 1))
    g3 = lax.broadcasted_iota(jnp.int32, (ROWSP, G, D), 1)
    x3 = lax.broadcast_in_dim(x, (ROWSP, G, D), (0, 2))
    big = jnp.where(b3 == g3, x3, -jnp.inf)
    mx[...] = jnp.maximum(mx[...], jnp.max(big, axis=0))

    @pl.when(i == pl.num_programs(0) - 1)
    def _():
        mean = sm[...] / jnp.maximum(cnt[...], 1.0)
        res = jnp.dot(mx[...], wf_ref[0:D, :], preferred_element_type=jnp.float32)
        res += jnp.dot(mean, wf_ref[D:2 * D, :], preferred_element_type=jnp.float32)
        out_ref[...] = res + bf_ref[...]


_pool = pl.pallas_call(
    _pool_body,
    grid=(N // ROWSP,),
    in_specs=[
        pl.BlockSpec((ROWSP, D), lambda i: (i, 0)),
        pl.BlockSpec((ROWSP, 1), lambda i: (i, 0)),
        pl.BlockSpec((2 * D, D), lambda i: (0, 0)),
        pl.BlockSpec((1, D), lambda i: (0, 0)),
    ],
    out_specs=pl.BlockSpec((G, D), lambda i: (0, 0)),
    out_shape=jax.ShapeDtypeStruct((G, D), jnp.float32),
    scratch_shapes=[
        pltpu.VMEM((G, D), jnp.float32),
        pltpu.VMEM((G, D), jnp.float32),
        pltpu.VMEM((G, 1), jnp.float32),
    ],
    compiler_params=pltpu.CompilerParams(dimension_semantics=("arbitrary",)),
)


def kernel(x, edge_index, batch, W1, b1, bn_g, bn_b, W2, b2, eps, ln_g, ln_b,
           Wf, bf):
    E = edge_index.shape[1]
    pad = EP - E
    src = jnp.concatenate([edge_index[0], jnp.zeros((pad,), jnp.int32)])
    dummy = N + jnp.arange(pad, dtype=jnp.int32) % (NACC - N)
    dst = jnp.concatenate([edge_index[1], dummy])
    srcp = src.reshape(EP // CH, CH)
    dstp = dst.reshape(EP // CH, CH)
    zeros_blk = jnp.zeros((ZROWS, D), jnp.float32)
    batch2 = batch.reshape(N, 1)

    out = x
    for l in range(W1.shape[0]):
        partials = _get_sc_aggregate()(out, srcp, dstp, zeros_blk)
        h1, st = _mlp1(eps[l].reshape(1, 1), out, partials[0], partials[1],
                       W1[l], b1[l].reshape(1, D))
        out = _mlp2(h1, st, bn_g[l].reshape(1, D), bn_b[l].reshape(1, D),
                    W2[l], b2[l].reshape(1, D), ln_g[l].reshape(1, D),
                    ln_b[l].reshape(1, D))
    return _pool(out, batch2, Wf, bf.reshape(1, D))


# speedup vs baseline: 2.9977x; 2.9910x over previous
import functools

import jax
import jax.numpy as jnp
from jax import lax
from jax.experimental import pallas as pl
from jax.experimental.pallas import tpu as pltpu
from jax.experimental.pallas import tpu_sc as plsc

N = 10000
D = 128
G = 64

CH = 128
PH = 40
NPHASE = 2
CHUNKS = PH * NPHASE
EP = 32 * CHUNKS * CH
NACC = 10240
ZROWS = 640
NBUF = 2


@functools.cache
def _get_sc_aggregate():
    mesh = plsc.VectorSubcoreMesh(core_axis_name="c", subcore_axis_name="s")

    @functools.partial(
        pl.kernel,
        out_type=jax.ShapeDtypeStruct((2, NACC, D), jnp.float32),
        scratch_types=[
            pltpu.VMEM((PH, CH), jnp.int32),
            pltpu.VMEM((PH, CH), jnp.int32),
            pltpu.VMEM((NBUF, CH, D), jnp.float32),
            pltpu.VMEM_SHARED((NACC, D), jnp.float32),
        ] + [pltpu.SemaphoreType.DMA] * NBUF,
        mesh=mesh,
    )
    def _sc_aggregate(x_hbm, srcp_hbm, dstp_hbm, zeros_hbm, out_hbm,
                      src_v, dst_v, rows_v, acc, *sems):
        c = lax.axis_index("c")
        s = lax.axis_index("s")

        pltpu.sync_copy(zeros_hbm, acc.at[pl.ds(s * ZROWS, ZROWS)])
        plsc.subcore_barrier()

        def run_phase(base):
            pltpu.sync_copy(srcp_hbm.at[pl.ds(base, PH)], src_v)
            pltpu.sync_copy(dstp_hbm.at[pl.ds(base, PH)], dst_v)

            for b in range(NBUF):
                pltpu.async_copy(x_hbm.at[src_v.at[b]], rows_v.at[b], sems[b])

            def ring_body(t, carry):
                for b in range(NBUF):
                    ci = t * NBUF + b
                    pltpu.make_async_copy(x_hbm.at[src_v.at[ci]],
                                          rows_v.at[b], sems[b]).wait()
                    pltpu.sync_copy(rows_v.at[b], acc.at[dst_v.at[ci]],
                                    add=True)
                    pltpu.async_copy(x_hbm.at[src_v.at[ci + NBUF]],
                                     rows_v.at[b], sems[b])
                return carry

            lax.fori_loop(0, PH // NBUF - 1, ring_body, 0)
            for b in range(NBUF):
                ci = PH - NBUF + b
                pltpu.make_async_copy(x_hbm.at[src_v.at[ci]], rows_v.at[b],
                                      sems[b]).wait()
                pltpu.sync_copy(rows_v.at[b], acc.at[dst_v.at[ci]], add=True)

        tid = c * 16 + s
        for p in range(NPHASE):
            run_phase(tid * CHUNKS + p * PH)

        plsc.subcore_barrier()
        pltpu.sync_copy(zeros_hbm.at[pl.ds(0, CH)], rows_v.at[0])
        pltpu.sync_copy(rows_v.at[0], acc.at[dst_v.at[PH - 1]], add=True)
        plsc.subcore_barrier()

        pltpu.sync_copy(acc.at[pl.ds(s * ZROWS, ZROWS)],
                        out_hbm.at[c, pl.ds(s * ZROWS, ZROWS)])

    return _sc_aggregate


ROWS1 = 1000


def _mlp1_body(eps_ref, x_ref, p0_ref, p1_ref, w1_ref, b1_ref, h_ref, st_ref):
    i = pl.program_id(0)
    a = x_ref[...] * (1.0 + eps_ref[0, 0]) + p0_ref[...] + p1_ref[...]
    h = jnp.dot(a, w1_ref[...], preferred_element_type=jnp.float32) + b1_ref[...]
    h_ref[...] = h
    s = jnp.sum(h, axis=0, keepdims=True)
    s2 = jnp.sum(h * h, axis=0, keepdims=True)

    @pl.when(i == 0)
    def _():
        st_ref[0:1, :] = s
        st_ref[1:2, :] = s2

    @pl.when(i > 0)
    def _():
        st_ref[0:1, :] += s
        st_ref[1:2, :] += s2


_mlp1 = pl.pallas_call(
    _mlp1_body,
    grid=(N // ROWS1,),
    in_specs=[
        pl.BlockSpec(memory_space=pltpu.SMEM),
        pl.BlockSpec((ROWS1, D), lambda i: (i, 0)),
        pl.BlockSpec((ROWS1, D), lambda i: (i, 0)),
        pl.BlockSpec((ROWS1, D), lambda i: (i, 0)),
        pl.BlockSpec((D, D), lambda i: (0, 0)),
        pl.BlockSpec((1, D), lambda i: (0, 0)),
    ],
    out_specs=[
        pl.BlockSpec((ROWS1, D), lambda i: (i, 0)),
        pl.BlockSpec((2, D), lambda i: (0, 0)),
    ],
    out_shape=[
        jax.ShapeDtypeStruct((N, D), jnp.float32),
        jax.ShapeDtypeStruct((2, D), jnp.float32),
    ],
    compiler_params=pltpu.CompilerParams(dimension_semantics=("arbitrary",)),
)


def _mlp2_body(h_ref, st_ref, bng_ref, bnb_ref, w2_ref, b2_ref, lng_ref,
               lnb_ref, out_ref):
    m = st_ref[0:1, :] / N
    v = st_ref[1:2, :] / N - m * m
    h = (h_ref[...] - m) * lax.rsqrt(v + 1e-5) * bng_ref[...] + bnb_ref[...]
    h = jnp.maximum(h, 0.0)
    h = jnp.dot(h, w2_ref[...], preferred_element_type=jnp.float32) + b2_ref[...]
    mu = jnp.mean(h, axis=1, keepdims=True)
    va = jnp.mean(h * h, axis=1, keepdims=True) - mu * mu
    h = (h - mu) * lax.rsqrt(va + 1e-5) * lng_ref[...] + lnb_ref[...]
    out_ref[...] = jnp.where(h > 0, h, 0.1 * h)


_mlp2 = pl.pallas_call(
    _mlp2_body,
    grid=(N // ROWS1,),
    in_specs=[
        pl.BlockSpec((ROWS1, D), lambda i: (i, 0)),
        pl.BlockSpec((2, D), lambda i: (0, 0)),
        pl.BlockSpec((1, D), lambda i: (0, 0)),
        pl.BlockSpec((1, D), lambda i: (0, 0)),
        pl.BlockSpec((D, D), lambda i: (0, 0)),
        pl.BlockSpec((1, D), lambda i: (0, 0)),
        pl.BlockSpec((1, D), lambda i: (0, 0)),
        pl.BlockSpec((1, D), lambda i: (0, 0)),
    ],
    out_specs=pl.BlockSpec((ROWS1, D), lambda i: (i, 0)),
    out_shape=jax.ShapeDtypeStruct((N, D), jnp.float32),
    compiler_params=pltpu.CompilerParams(dimension_semantics=("arbitrary",)),
)


ROWSP = 200


def _pool_body(x_ref, b_ref, wf_ref, bf_ref, out_ref, mx, sm, cnt):
    i = pl.program_id(0)

    @pl.when(i == 0)
    def _():
        mx[...] = jnp.full((G, D), -jnp.inf, jnp.float32)
        sm[...] = jnp.zeros((G, D), jnp.float32)
        cnt[...] = jnp.zeros((G, 1), jnp.float32)

    x = x_ref[...]
    gids = lax.broadcasted_iota(jnp.int32, (1, G), 1)
    oh = (b_ref[...] == gids).astype(jnp.float32)
    dn = (((0,), (0,)), ((), ()))
    sm[...] += lax.dot_general(oh, x, dn, preferred_element_type=jnp.float32)
    cnt[...] += lax.dot_general(oh, jnp.ones((ROWSP, 1), jnp.float32), dn,
                                preferred_element_type=jnp.float32)
    b3 = lax.broadcast_in_dim(b_ref[...], (ROWSP, G, D), (0, 1))
    g3 = lax.broadcasted_iota(jnp.int32, (ROWSP, G, D), 1)
    x3 = lax.broadcast_in_dim(x, (ROWSP, G, D), (0, 2))
    big = jnp.where(b3 == g3, x3, -jnp.inf)
    mx[...] = jnp.maximum(mx[...], jnp.max(big, axis=0))

    @pl.when(i == pl.num_programs(0) - 1)
    def _():
        mean = sm[...] / jnp.maximum(cnt[...], 1.0)
        res = jnp.dot(mx[...], wf_ref[0:D, :], preferred_element_type=jnp.float32)
        res += jnp.dot(mean, wf_ref[D:2 * D, :], preferred_element_type=jnp.float32)
        out_ref[...] = res + bf_ref[...]


_pool = pl.pallas_call(
    _pool_body,
    grid=(N // ROWSP,),
    in_specs=[
        pl.BlockSpec((ROWSP, D), lambda i: (i, 0)),
        pl.BlockSpec((ROWSP, 1), lambda i: (i, 0)),
        pl.BlockSpec((2 * D, D), lambda i: (0, 0)),
        pl.BlockSpec((1, D), lambda i: (0, 0)),
    ],
    out_specs=pl.BlockSpec((G, D), lambda i: (0, 0)),
    out_shape=jax.ShapeDtypeStruct((G, D), jnp.float32),
    scratch_shapes=[
        pltpu.VMEM((G, D), jnp.float32),
        pltpu.VMEM((G, D), jnp.float32),
        pltpu.VMEM((G, 1), jnp.float32),
    ],
    compiler_params=pltpu.CompilerParams(dimension_semantics=("arbitrary",)),
)


def kernel(x, edge_index, batch, W1, b1, bn_g, bn_b, W2, b2, eps, ln_g, ln_b,
           Wf, bf):
    E = edge_index.shape[1]
    pad = EP - E
    dsrc = jnp.arange(pad, dtype=jnp.int32) * 8 % N
    src = jnp.concatenate([edge_index[0], dsrc])
    dummy = N + jnp.arange(pad, dtype=jnp.int32) % (NACC - N)
    dst = jnp.concatenate([edge_index[1], dummy])
    srcp = src.reshape(EP // CH, CH)
    dstp = dst.reshape(EP // CH, CH)
    zeros_blk = jnp.zeros((ZROWS, D), jnp.float32)
    batch2 = batch.reshape(N, 1)

    out = x
    for l in range(W1.shape[0]):
        partials = _get_sc_aggregate()(out, srcp, dstp, zeros_blk)
        h1, st = _mlp1(eps[l].reshape(1, 1), out, partials[0], partials[1],
                       W1[l], b1[l].reshape(1, D))
        out = _mlp2(h1, st, bn_g[l].reshape(1, D), bn_b[l].reshape(1, D),
                    W2[l], b2[l].reshape(1, D), ln_g[l].reshape(1, D),
                    ln_b[l].reshape(1, D))
    return _pool(out, batch2, Wf, bf.reshape(1, D))


# speedup vs baseline: 3.1361x; 1.0462x over previous
import functools

import jax
import jax.numpy as jnp
from jax import lax
from jax.experimental import pallas as pl
from jax.experimental.pallas import tpu as pltpu
from jax.experimental.pallas import tpu_sc as plsc

N = 10000
D = 128
G = 64

CH = 64
PH = 40
NPHASE = 4
CHUNKS = PH * NPHASE
EP = 32 * CHUNKS * CH
NACC = 10240
ZROWS = 640
NBUF = 4


@functools.cache
def _get_sc_aggregate():
    mesh = plsc.VectorSubcoreMesh(core_axis_name="c", subcore_axis_name="s")

    @functools.partial(
        pl.kernel,
        out_type=jax.ShapeDtypeStruct((2, NACC, D), jnp.float32),
        scratch_types=[
            pltpu.VMEM((PH, CH), jnp.int32),
            pltpu.VMEM((PH, CH), jnp.int32),
            pltpu.VMEM((NBUF, CH, D), jnp.float32),
            pltpu.VMEM_SHARED((NACC, D), jnp.float32),
        ] + [pltpu.SemaphoreType.DMA] * NBUF,
        mesh=mesh,
    )
    def _sc_aggregate(x_hbm, srcp_hbm, dstp_hbm, zeros_hbm, out_hbm,
                      src_v, dst_v, rows_v, acc, *sems):
        c = lax.axis_index("c")
        s = lax.axis_index("s")

        pltpu.sync_copy(zeros_hbm, acc.at[pl.ds(s * ZROWS, ZROWS)])
        plsc.subcore_barrier()

        def run_phase(base):
            pltpu.sync_copy(srcp_hbm.at[pl.ds(base, PH)], src_v)
            pltpu.sync_copy(dstp_hbm.at[pl.ds(base, PH)], dst_v)

            for b in range(NBUF):
                pltpu.async_copy(x_hbm.at[src_v.at[b]], rows_v.at[b], sems[b])

            def ring_body(t, carry):
                for b in range(NBUF):
                    ci = t * NBUF + b
                    pltpu.make_async_copy(x_hbm.at[src_v.at[ci]],
                                          rows_v.at[b], sems[b]).wait()
                    pltpu.sync_copy(rows_v.at[b], acc.at[dst_v.at[ci]],
                                    add=True)
                    pltpu.async_copy(x_hbm.at[src_v.at[ci + NBUF]],
                                     rows_v.at[b], sems[b])
                return carry

            lax.fori_loop(0, PH // NBUF - 1, ring_body, 0)
            for b in range(NBUF):
                ci = PH - NBUF + b
                pltpu.make_async_copy(x_hbm.at[src_v.at[ci]], rows_v.at[b],
                                      sems[b]).wait()
                pltpu.sync_copy(rows_v.at[b], acc.at[dst_v.at[ci]], add=True)

        tid = c * 16 + s
        for p in range(NPHASE):
            run_phase(tid * CHUNKS + p * PH)

        plsc.subcore_barrier()
        pltpu.sync_copy(zeros_hbm.at[pl.ds(0, CH)], rows_v.at[0])
        pltpu.sync_copy(rows_v.at[0], acc.at[dst_v.at[PH - 1]], add=True)
        plsc.subcore_barrier()

        pltpu.sync_copy(acc.at[pl.ds(s * ZROWS, ZROWS)],
                        out_hbm.at[c, pl.ds(s * ZROWS, ZROWS)])

    return _sc_aggregate


ROWS1 = 1000


def _mlp1_body(eps_ref, x_ref, p0_ref, p1_ref, w1_ref, b1_ref, h_ref, st_ref):
    i = pl.program_id(0)
    a = x_ref[...] * (1.0 + eps_ref[0, 0]) + p0_ref[...] + p1_ref[...]
    h = jnp.dot(a, w1_ref[...], preferred_element_type=jnp.float32) + b1_ref[...]
    h_ref[...] = h
    s = jnp.sum(h, axis=0, keepdims=True)
    s2 = jnp.sum(h * h, axis=0, keepdims=True)

    @pl.when(i == 0)
    def _():
        st_ref[0:1, :] = s
        st_ref[1:2, :] = s2

    @pl.when(i > 0)
    def _():
        st_ref[0:1, :] += s
        st_ref[1:2, :] += s2


_mlp1 = pl.pallas_call(
    _mlp1_body,
    grid=(N // ROWS1,),
    in_specs=[
        pl.BlockSpec(memory_space=pltpu.SMEM),
        pl.BlockSpec((ROWS1, D), lambda i: (i, 0)),
        pl.BlockSpec((ROWS1, D), lambda i: (i, 0)),
        pl.BlockSpec((ROWS1, D), lambda i: (i, 0)),
        pl.BlockSpec((D, D), lambda i: (0, 0)),
        pl.BlockSpec((1, D), lambda i: (0, 0)),
    ],
    out_specs=[
        pl.BlockSpec((ROWS1, D), lambda i: (i, 0)),
        pl.BlockSpec((2, D), lambda i: (0, 0)),
    ],
    out_shape=[
        jax.ShapeDtypeStruct((N, D), jnp.float32),
        jax.ShapeDtypeStruct((2, D), jnp.float32),
    ],
    compiler_params=pltpu.CompilerParams(dimension_semantics=("arbitrary",)),
)


def _mlp2_body(h_ref, st_ref, bng_ref, bnb_ref, w2_ref, b2_ref, lng_ref,
               lnb_ref, out_ref):
    m = st_ref[0:1, :] / N
    v = st_ref[1:2, :] / N - m * m
    h = (h_ref[...] - m) * lax.rsqrt(v + 1e-5) * bng_ref[...] + bnb_ref[...]
    h = jnp.maximum(h, 0.0)
    h = jnp.dot(h, w2_ref[...], preferred_element_type=jnp.float32) + b2_ref[...]
    mu = jnp.mean(h, axis=1, keepdims=True)
    va = jnp.mean(h * h, axis=1, keepdims=True) - mu * mu
    h = (h - mu) * lax.rsqrt(va + 1e-5) * lng_ref[...] + lnb_ref[...]
    out_ref[...] = jnp.where(h > 0, h, 0.1 * h)


_mlp2 = pl.pallas_call(
    _mlp2_body,
    grid=(N // ROWS1,),
    in_specs=[
        pl.BlockSpec((ROWS1, D), lambda i: (i, 0)),
        pl.BlockSpec((2, D), lambda i: (0, 0)),
        pl.BlockSpec((1, D), lambda i: (0, 0)),
        pl.BlockSpec((1, D), lambda i: (0, 0)),
        pl.BlockSpec((D, D), lambda i: (0, 0)),
        pl.BlockSpec((1, D), lambda i: (0, 0)),
        pl.BlockSpec((1, D), lambda i: (0, 0)),
        pl.BlockSpec((1, D), lambda i: (0, 0)),
    ],
    out_specs=pl.BlockSpec((ROWS1, D), lambda i: (i, 0)),
    out_shape=jax.ShapeDtypeStruct((N, D), jnp.float32),
    compiler_params=pltpu.CompilerParams(dimension_semantics=("arbitrary",)),
)


ROWSP = 200


def _pool_body(x_ref, b_ref, wf_ref, bf_ref, out_ref, mx, sm, cnt):
    i = pl.program_id(0)

    @pl.when(i == 0)
    def _():
        mx[...] = jnp.full((G, D), -jnp.inf, jnp.float32)
        sm[...] = jnp.zeros((G, D), jnp.float32)
        cnt[...] = jnp.zeros((G, 1), jnp.float32)

    x = x_ref[...]
    gids = lax.broadcasted_iota(jnp.int32, (1, G), 1)
    oh = (b_ref[...] == gids).astype(jnp.float32)
    dn = (((0,), (0,)), ((), ()))
    sm[...] += lax.dot_general(oh, x, dn, preferred_element_type=jnp.float32)
    cnt[...] += lax.dot_general(oh, jnp.ones((ROWSP, 1), jnp.float32), dn,
                                preferred_element_type=jnp.float32)
    b3 = lax.broadcast_in_dim(b_ref[...], (ROWSP, G, D), (0, 1))
    g3 = lax.broadcasted_iota(jnp.int32, (ROWSP, G, D), 1)
    x3 = lax.broadcast_in_dim(x, (ROWSP, G, D), (0, 2))
    big = jnp.where(b3 == g3, x3, -jnp.inf)
    mx[...] = jnp.maximum(mx[...], jnp.max(big, axis=0))

    @pl.when(i == pl.num_programs(0) - 1)
    def _():
        mean = sm[...] / jnp.maximum(cnt[...], 1.0)
        res = jnp.dot(mx[...], wf_ref[0:D, :], preferred_element_type=jnp.float32)
        res += jnp.dot(mean, wf_ref[D:2 * D, :], preferred_element_type=jnp.float32)
        out_ref[...] = res + bf_ref[...]


_pool = pl.pallas_call(
    _pool_body,
    grid=(N // ROWSP,),
    in_specs=[
        pl.BlockSpec((ROWSP, D), lambda i: (i, 0)),
        pl.BlockSpec((ROWSP, 1), lambda i: (i, 0)),
        pl.BlockSpec((2 * D, D), lambda i: (0, 0)),
        pl.BlockSpec((1, D), lambda i: (0, 0)),
    ],
    out_specs=pl.BlockSpec((G, D), lambda i: (0, 0)),
    out_shape=jax.ShapeDtypeStruct((G, D), jnp.float32),
    scratch_shapes=[
        pltpu.VMEM((G, D), jnp.float32),
        pltpu.VMEM((G, D), jnp.float32),
        pltpu.VMEM((G, 1), jnp.float32),
    ],
    compiler_params=pltpu.CompilerParams(dimension_semantics=("arbitrary",)),
)


def kernel(x, edge_index, batch, W1, b1, bn_g, bn_b, W2, b2, eps, ln_g, ln_b,
           Wf, bf):
    E = edge_index.shape[1]
    pad = EP - E
    dsrc = jnp.arange(pad, dtype=jnp.int32) * 8 % N
    src = jnp.concatenate([edge_index[0], dsrc])
    dummy = N + jnp.arange(pad, dtype=jnp.int32) % (NACC - N)
    dst = jnp.concatenate([edge_index[1], dummy])
    srcp = src.reshape(EP // CH, CH)
    dstp = dst.reshape(EP // CH, CH)
    zeros_blk = jnp.zeros((ZROWS, D), jnp.float32)
    batch2 = batch.reshape(N, 1)

    out = x
    for l in range(W1.shape[0]):
        partials = _get_sc_aggregate()(out, srcp, dstp, zeros_blk)
        h1, st = _mlp1(eps[l].reshape(1, 1), out, partials[0], partials[1],
                       W1[l], b1[l].reshape(1, D))
        out = _mlp2(h1, st, bn_g[l].reshape(1, D), bn_b[l].reshape(1, D),
                    W2[l], b2[l].reshape(1, D), ln_g[l].reshape(1, D),
                    ln_b[l].reshape(1, D))
    return _pool(out, batch2, Wf, bf.reshape(1, D))


# speedup vs baseline: 3.2085x; 1.0231x over previous
import functools

import jax
import jax.numpy as jnp
from jax import lax
from jax.experimental import pallas as pl
from jax.experimental.pallas import tpu as pltpu
from jax.experimental.pallas import tpu_sc as plsc

N = 10000
D = 128
G = 64

CH = 64
PH = 40
NPHASE = 4
CHUNKS = PH * NPHASE
EP = 32 * CHUNKS * CH
NACC = 10240
ZROWS = 640
NBUF = 4


@functools.cache
def _get_sc_aggregate():
    mesh = plsc.VectorSubcoreMesh(core_axis_name="c", subcore_axis_name="s")

    @functools.partial(
        pl.kernel,
        out_type=jax.ShapeDtypeStruct((2, NACC, D), jnp.float32),
        scratch_types=[
            pltpu.VMEM((PH, CH), jnp.int32),
            pltpu.VMEM((PH, CH), jnp.int32),
            pltpu.VMEM((NBUF, CH, D), jnp.float32),
            pltpu.VMEM_SHARED((NACC, D), jnp.float32),
        ] + [pltpu.SemaphoreType.DMA] * NBUF,
        mesh=mesh,
    )
    def _sc_aggregate(x_hbm, srcp_hbm, dstp_hbm, zeros_hbm, out_hbm,
                      src_v, dst_v, rows_v, acc, *sems):
        c = lax.axis_index("c")
        s = lax.axis_index("s")

        pltpu.sync_copy(zeros_hbm, acc.at[pl.ds(s * ZROWS, ZROWS)])
        plsc.subcore_barrier()

        def run_phase(base):
            pltpu.sync_copy(srcp_hbm.at[pl.ds(base, PH)], src_v)
            pltpu.sync_copy(dstp_hbm.at[pl.ds(base, PH)], dst_v)

            for b in range(NBUF):
                pltpu.async_copy(x_hbm.at[src_v.at[b]], rows_v.at[b], sems[b])

            def ring_body(t, carry):
                for b in range(NBUF):
                    ci = t * NBUF + b
                    pltpu.make_async_copy(x_hbm.at[src_v.at[ci]],
                                          rows_v.at[b], sems[b]).wait()
                    pltpu.sync_copy(rows_v.at[b], acc.at[dst_v.at[ci]],
                                    add=True)
                    pltpu.async_copy(x_hbm.at[src_v.at[ci + NBUF]],
                                     rows_v.at[b], sems[b])
                return carry

            lax.fori_loop(0, PH // NBUF - 1, ring_body, 0)
            for b in range(NBUF):
                ci = PH - NBUF + b
                pltpu.make_async_copy(x_hbm.at[src_v.at[ci]], rows_v.at[b],
                                      sems[b]).wait()
                pltpu.sync_copy(rows_v.at[b], acc.at[dst_v.at[ci]], add=True)

        tid = c * 16 + s
        for p in range(NPHASE):
            run_phase(tid * CHUNKS + p * PH)

        plsc.subcore_barrier()
        pltpu.sync_copy(zeros_hbm.at[pl.ds(0, CH)], rows_v.at[0])
        pltpu.sync_copy(rows_v.at[0], acc.at[dst_v.at[PH - 1]], add=True)
        plsc.subcore_barrier()

        pltpu.sync_copy(acc.at[pl.ds(s * ZROWS, ZROWS)],
                        out_hbm.at[c, pl.ds(s * ZROWS, ZROWS)])

    return _sc_aggregate


ROWS1 = 1000
NBLK = N // ROWS1


def _make_mlp(l):
    def body(eps_ref, x_ref, p0_ref, p1_ref, w1_ref, b1_ref, bng_ref,
             bnb_ref, w2_ref, b2_ref, lng_ref, lnb_ref, out_ref, h1_ref,
             st_ref):
        p = pl.program_id(0)
        i = pl.program_id(1)

        @pl.when(p == 0)
        def _():
            a = (x_ref[...] * (1.0 + eps_ref[l]) + p0_ref[...] + p1_ref[...])
            h = jnp.dot(a, w1_ref[0], preferred_element_type=jnp.float32)
            h += b1_ref[0]
            h1_ref[pl.ds(i * ROWS1, ROWS1), :] = h
            su = jnp.sum(h, axis=0, keepdims=True)
            s2 = jnp.sum(h * h, axis=0, keepdims=True)

            @pl.when(i == 0)
            def _():
                st_ref[0:1, :] = su
                st_ref[1:2, :] = s2

            @pl.when(i > 0)
            def _():
                st_ref[0:1, :] += su
                st_ref[1:2, :] += s2

        @pl.when(p == 1)
        def _():
            m = st_ref[0:1, :] / N
            v = st_ref[1:2, :] / N - m * m
            h = h1_ref[pl.ds(i * ROWS1, ROWS1), :]
            h = (h - m) * lax.rsqrt(v + 1e-5) * bng_ref[0] + bnb_ref[0]
            h = jnp.maximum(h, 0.0)
            h = jnp.dot(h, w2_ref[0], preferred_element_type=jnp.float32)
            h += b2_ref[0]
            mu = jnp.mean(h, axis=1, keepdims=True)
            va = jnp.mean(h * h, axis=1, keepdims=True) - mu * mu
            h = (h - mu) * lax.rsqrt(va + 1e-5) * lng_ref[0] + lnb_ref[0]
            out_ref[...] = jnp.where(h > 0, h, 0.1 * h)

    row_spec = pl.BlockSpec((ROWS1, D), lambda p, i: (i * (1 - p), 0))
    lay2 = pl.BlockSpec((1, 1, D), lambda p, i: (l, 0, 0))
    lay3 = pl.BlockSpec((1, D, D), lambda p, i: (l, 0, 0))
    return pl.pallas_call(
        body,
        grid=(2, NBLK),
        in_specs=[
            pl.BlockSpec(memory_space=pltpu.SMEM),
            row_spec, row_spec, row_spec,
            lay3, lay2, lay2, lay2, lay3, lay2, lay2, lay2,
        ],
        out_specs=pl.BlockSpec((ROWS1, D), lambda p, i: (i * p, 0)),
        out_shape=jax.ShapeDtypeStruct((N, D), jnp.float32),
        scratch_shapes=[
            pltpu.VMEM((N, D), jnp.float32),
            pltpu.VMEM((2, D), jnp.float32),
        ],
        compiler_params=pltpu.CompilerParams(
            dimension_semantics=("arbitrary", "arbitrary")),
    )


ROWSP = 200


def _pool_body(x_ref, b_ref, wf_ref, bf_ref, out_ref, mx, sm, cnt):
    i = pl.program_id(0)

    @pl.when(i == 0)
    def _():
        mx[...] = jnp.full((G, D), -jnp.inf, jnp.float32)
        sm[...] = jnp.zeros((G, D), jnp.float32)
        cnt[...] = jnp.zeros((G, 1), jnp.float32)

    x = x_ref[...]
    gids = lax.broadcasted_iota(jnp.int32, (1, G), 1)
    oh = (b_ref[...] == gids).astype(jnp.float32)
    dn = (((0,), (0,)), ((), ()))
    sm[...] += lax.dot_general(oh, x, dn, preferred_element_type=jnp.float32)
    cnt[...] += lax.dot_general(oh, jnp.ones((ROWSP, 1), jnp.float32), dn,
                                preferred_element_type=jnp.float32)
    b3 = lax.broadcast_in_dim(b_ref[...], (ROWSP, G, D), (0, 1))
    g3 = lax.broadcasted_iota(jnp.int32, (ROWSP, G, D), 1)
    x3 = lax.broadcast_in_dim(x, (ROWSP, G, D), (0, 2))
    big = jnp.where(b3 == g3, x3, -jnp.inf)
    mx[...] = jnp.maximum(mx[...], jnp.max(big, axis=0))

    @pl.when(i == pl.num_programs(0) - 1)
    def _():
        mean = sm[...] / jnp.maximum(cnt[...], 1.0)
        res = jnp.dot(mx[...], wf_ref[0:D, :], preferred_element_type=jnp.float32)
        res += jnp.dot(mean, wf_ref[D:2 * D, :], preferred_element_type=jnp.float32)
        out_ref[...] = res + bf_ref[...]


_pool = pl.pallas_call(
    _pool_body,
    grid=(N // ROWSP,),
    in_specs=[
        pl.BlockSpec((ROWSP, D), lambda i: (i, 0)),
        pl.BlockSpec((ROWSP, 1), lambda i: (i, 0)),
        pl.BlockSpec((2 * D, D), lambda i: (0, 0)),
        pl.BlockSpec((1, D), lambda i: (0, 0)),
    ],
    out_specs=pl.BlockSpec((G, D), lambda i: (0, 0)),
    out_shape=jax.ShapeDtypeStruct((G, D), jnp.float32),
    scratch_shapes=[
        pltpu.VMEM((G, D), jnp.float32),
        pltpu.VMEM((G, D), jnp.float32),
        pltpu.VMEM((G, 1), jnp.float32),
    ],
    compiler_params=pltpu.CompilerParams(dimension_semantics=("arbitrary",)),
)


def kernel(x, edge_index, batch, W1, b1, bn_g, bn_b, W2, b2, eps, ln_g, ln_b,
           Wf, bf):
    E = edge_index.shape[1]
    pad = EP - E
    dsrc = jnp.arange(pad, dtype=jnp.int32) * 8 % N
    src = jnp.concatenate([edge_index[0], dsrc])
    dummy = N + jnp.arange(pad, dtype=jnp.int32) % (NACC - N)
    dst = jnp.concatenate([edge_index[1], dummy])
    srcp = src.reshape(EP // CH, CH)
    dstp = dst.reshape(EP // CH, CH)
    zeros_blk = jnp.zeros((ZROWS, D), jnp.float32)
    batch2 = batch.reshape(N, 1)

    L = W1.shape[0]
    b1r, bn_gr, bn_br = (a.reshape(L, 1, D) for a in (b1, bn_g, bn_b))
    b2r, ln_gr, ln_br = (a.reshape(L, 1, D) for a in (b2, ln_g, ln_b))
    out = x
    for l in range(L):
        partials = _get_sc_aggregate()(out, srcp, dstp, zeros_blk)
        out = _make_mlp(l)(eps, out, partials[0], partials[1],
                           W1, b1r, bn_gr, bn_br, W2, b2r, ln_gr, ln_br)
    return _pool(out, batch2, Wf, bf.reshape(1, D))


# speedup vs baseline: 3.2607x; 1.0163x over previous
import functools

import jax
import jax.numpy as jnp
from jax import lax
from jax.experimental import pallas as pl
from jax.experimental.pallas import tpu as pltpu
from jax.experimental.pallas import tpu_sc as plsc

N = 10000
D = 128
G = 64

CH = 50
PH = 40
NPHASE = 5
CHUNKS = PH * NPHASE
EP = 32 * CHUNKS * CH
NACC = 10240
ZROWS = 640
NBUF = 4
FL = 32


@functools.cache
def _get_sc_aggregate():
    mesh = plsc.VectorSubcoreMesh(core_axis_name="c", subcore_axis_name="s")

    @functools.partial(
        pl.kernel,
        out_type=jax.ShapeDtypeStruct((2, NACC, D), jnp.float32),
        scratch_types=[
            pltpu.VMEM((PH, CH), jnp.int32),
            pltpu.VMEM((PH, CH), jnp.int32),
            pltpu.VMEM((NBUF, CH, D), jnp.float32),
            pltpu.VMEM((FL, D), jnp.float32),
            pltpu.VMEM((FL,), jnp.int32),
            pltpu.VMEM_SHARED((NACC, D), jnp.float32),
        ] + [pltpu.SemaphoreType.DMA] * NBUF,
        mesh=mesh,
    )
    def _sc_aggregate(x_hbm, srcp_hbm, dstp_hbm, zeros_hbm, fidx_hbm, out_hbm,
                      src_v, dst_v, rows_v, zb_v, fidx_v, acc, *sems):
        c = lax.axis_index("c")
        s = lax.axis_index("s")

        pltpu.sync_copy(zeros_hbm, acc.at[pl.ds(s * ZROWS, ZROWS)])
        plsc.subcore_barrier()

        def run_phase(base):
            pltpu.sync_copy(srcp_hbm.at[pl.ds(base, PH)], src_v)
            pltpu.sync_copy(dstp_hbm.at[pl.ds(base, PH)], dst_v)

            for b in range(NBUF):
                pltpu.async_copy(x_hbm.at[src_v.at[b]], rows_v.at[b], sems[b])

            def ring_body(t, carry):
                for b in range(NBUF):
                    ci = t * NBUF + b
                    pltpu.make_async_copy(x_hbm.at[src_v.at[ci]],
                                          rows_v.at[b], sems[b]).wait()
                    pltpu.sync_copy(rows_v.at[b], acc.at[dst_v.at[ci]],
                                    add=True)
                    pltpu.async_copy(x_hbm.at[src_v.at[ci + NBUF]],
                                     rows_v.at[b], sems[b])
                return carry

            lax.fori_loop(0, PH // NBUF - 1, ring_body, 0)
            for b in range(NBUF):
                ci = PH - NBUF + b
                pltpu.make_async_copy(x_hbm.at[src_v.at[ci]], rows_v.at[b],
                                      sems[b]).wait()
                pltpu.sync_copy(rows_v.at[b], acc.at[dst_v.at[ci]], add=True)

        tid = c * 16 + s
        for p in range(NPHASE):
            run_phase(tid * CHUNKS + p * PH)

        plsc.subcore_barrier()
        pltpu.sync_copy(zeros_hbm.at[pl.ds(0, FL)], zb_v)
        pltpu.sync_copy(fidx_hbm, fidx_v)
        pltpu.sync_copy(zb_v, acc.at[fidx_v], add=True)
        plsc.subcore_barrier()

        pltpu.sync_copy(acc.at[pl.ds(s * ZROWS, ZROWS)],
                        out_hbm.at[c, pl.ds(s * ZROWS, ZROWS)])

    return _sc_aggregate


ROWS1 = 1000
NBLK = N // ROWS1


def _make_mlp(l):
    def body(eps_ref, x_ref, p0_ref, p1_ref, w1_ref, b1_ref, bng_ref,
             bnb_ref, w2_ref, b2_ref, lng_ref, lnb_ref, out_ref, h1_ref,
             st_ref):
        p = pl.program_id(0)
        i = pl.program_id(1)

        @pl.when(p == 0)
        def _():
            a = (x_ref[...] * (1.0 + eps_ref[l]) + p0_ref[0] + p1_ref[0])
            h = jnp.dot(a, w1_ref[0], preferred_element_type=jnp.float32)
            h += b1_ref[0]
            h1_ref[pl.ds(i * ROWS1, ROWS1), :] = h
            su = jnp.sum(h, axis=0, keepdims=True)
            s2 = jnp.sum(h * h, axis=0, keepdims=True)

            @pl.when(i == 0)
            def _():
                st_ref[0:1, :] = su
                st_ref[1:2, :] = s2

            @pl.when(i > 0)
            def _():
                st_ref[0:1, :] += su
                st_ref[1:2, :] += s2

        @pl.when(p == 1)
        def _():
            m = st_ref[0:1, :] / N
            v = st_ref[1:2, :] / N - m * m
            h = h1_ref[pl.ds(i * ROWS1, ROWS1), :]
            h = (h - m) * lax.rsqrt(v + 1e-5) * bng_ref[0] + bnb_ref[0]
            h = jnp.maximum(h, 0.0)
            h = jnp.dot(h, w2_ref[0], preferred_element_type=jnp.float32)
            h += b2_ref[0]
            mu = jnp.mean(h, axis=1, keepdims=True)
            va = jnp.mean(h * h, axis=1, keepdims=True) - mu * mu
            h = (h - mu) * lax.rsqrt(va + 1e-5) * lng_ref[0] + lnb_ref[0]
            out_ref[...] = jnp.where(h > 0, h, 0.1 * h)

    row_spec = pl.BlockSpec((ROWS1, D), lambda p, i: (i * (1 - p), 0))
    par0 = pl.BlockSpec((1, ROWS1, D), lambda p, i: (0, i * (1 - p), 0))
    par1 = pl.BlockSpec((1, ROWS1, D), lambda p, i: (1, i * (1 - p), 0))
    lay2 = pl.BlockSpec((1, 1, D), lambda p, i: (l, 0, 0))
    lay3 = pl.BlockSpec((1, D, D), lambda p, i: (l, 0, 0))
    return pl.pallas_call(
        body,
        grid=(2, NBLK),
        in_specs=[
            pl.BlockSpec(memory_space=pltpu.SMEM),
            row_spec, par0, par1,
            lay3, lay2, lay2, lay2, lay3, lay2, lay2, lay2,
        ],
        out_specs=pl.BlockSpec((ROWS1, D), lambda p, i: (i * p, 0)),
        out_shape=jax.ShapeDtypeStruct((N, D), jnp.float32),
        scratch_shapes=[
            pltpu.VMEM((N, D), jnp.float32),
            pltpu.VMEM((2, D), jnp.float32),
        ],
        compiler_params=pltpu.CompilerParams(
            dimension_semantics=("arbitrary", "arbitrary")),
    )


ROWSP = 200


def _pool_body(x_ref, b_ref, wf_ref, bf_ref, out_ref, mx, sm, cnt):
    i = pl.program_id(0)

    @pl.when(i == 0)
    def _():
        mx[...] = jnp.full((G, D), -jnp.inf, jnp.float32)
        sm[...] = jnp.zeros((G, D), jnp.float32)
        cnt[...] = jnp.zeros((G, 1), jnp.float32)

    x = x_ref[...]
    gids = lax.broadcasted_iota(jnp.int32, (1, G), 1)
    oh = (b_ref[...] == gids).astype(jnp.float32)
    dn = (((0,), (0,)), ((), ()))
    sm[...] += lax.dot_general(oh, x, dn, preferred_element_type=jnp.float32)
    cnt[...] += lax.dot_general(oh, jnp.ones((ROWSP, 1), jnp.float32), dn,
                                preferred_element_type=jnp.float32)
    b3 = lax.broadcast_in_dim(b_ref[...], (ROWSP, G, D), (0, 1))
    g3 = lax.broadcasted_iota(jnp.int32, (ROWSP, G, D), 1)
    x3 = lax.broadcast_in_dim(x, (ROWSP, G, D), (0, 2))
    big = jnp.where(b3 == g3, x3, -jnp.inf)
    mx[...] = jnp.maximum(mx[...], jnp.max(big, axis=0))

    @pl.when(i == pl.num_programs(0) - 1)
    def _():
        mean = sm[...] / jnp.maximum(cnt[...], 1.0)
        res = jnp.dot(mx[...], wf_ref[0:D, :], preferred_element_type=jnp.float32)
        res += jnp.dot(mean, wf_ref[D:2 * D, :], preferred_element_type=jnp.float32)
        out_ref[...] = res + bf_ref[...]


_pool = pl.pallas_call(
    _pool_body,
    grid=(N // ROWSP,),
    in_specs=[
        pl.BlockSpec((ROWSP, D), lambda i: (i, 0)),
        pl.BlockSpec((ROWSP, 1), lambda i: (i, 0)),
        pl.BlockSpec((2 * D, D), lambda i: (0, 0)),
        pl.BlockSpec((1, D), lambda i: (0, 0)),
    ],
    out_specs=pl.BlockSpec((G, D), lambda i: (0, 0)),
    out_shape=jax.ShapeDtypeStruct((G, D), jnp.float32),
    scratch_shapes=[
        pltpu.VMEM((G, D), jnp.float32),
        pltpu.VMEM((G, D), jnp.float32),
        pltpu.VMEM((G, 1), jnp.float32),
    ],
    compiler_params=pltpu.CompilerParams(dimension_semantics=("arbitrary",)),
)


def kernel(x, edge_index, batch, W1, b1, bn_g, bn_b, W2, b2, eps, ln_g, ln_b,
           Wf, bf):
    srcp = edge_index[0].reshape(EP // CH, CH)
    dstp = edge_index[1].reshape(EP // CH, CH)
    zeros_blk = jnp.zeros((ZROWS, D), jnp.float32)
    fidx = jnp.arange(FL, dtype=jnp.int32)
    batch2 = batch.reshape(N, 1)

    L = W1.shape[0]
    b1r, bn_gr, bn_br = (a.reshape(L, 1, D) for a in (b1, bn_g, bn_b))
    b2r, ln_gr, ln_br = (a.reshape(L, 1, D) for a in (b2, ln_g, ln_b))
    out = x
    for l in range(L):
        partials = _get_sc_aggregate()(out, srcp, dstp, zeros_blk, fidx)
        out = _make_mlp(l)(eps, out, partials, partials,
                           W1, b1r, bn_gr, bn_br, W2, b2r, ln_gr, ln_br)
    return _pool(out, batch2, Wf, bf.reshape(1, D))
